# baseline (speedup 1.0000x reference)
import functools

import jax
import jax.numpy as jnp
from jax.experimental import pallas as pl
from jax.experimental.pallas import tpu as pltpu

_NUM_ITERS = 10


def _assign_kernel(sums_ref, counts_ref, x_ref, out_ref, cnt_out_ref,
                   centers_s, c2_s, cnt_s, *, K, C, Pb, n_b, n_pb):
    pb = pl.program_id(1)

    @pl.when(pb == 0)
    def _compute_centers():
        cnt = counts_ref[...]
        sums = sums_ref[...]
        centers = jnp.where(cnt > 0.0,
                            sums / jnp.maximum(cnt, 1.0),
                            jnp.zeros_like(sums))
        centers_s[...] = centers
        c2_s[...] = jnp.sum(centers * centers, axis=1, keepdims=True)
        cnt_s[...] = jnp.zeros_like(cnt_s)

    centers = centers_s[...]
    x = x_ref[0]

    f2 = jnp.sum(x * x, axis=0, keepdims=True)
    cross = jnp.dot(centers, x, preferred_element_type=jnp.float32)
    d2 = f2 - 2.0 * cross + c2_s[...]

    minv = jnp.min(d2, axis=0, keepdims=True)
    kiota = jax.lax.broadcasted_iota(jnp.int32, (K, Pb), 0)
    idx = jnp.min(jnp.where(d2 == minv, kiota, K), axis=0, keepdims=True)
    out_ref[...] = idx.reshape(1, 1, Pb)

    onehot = (kiota == idx).astype(jnp.float32)
    cnt_s[...] += jnp.sum(onehot, axis=1, keepdims=True)

    @pl.when(pb == n_pb - 1)
    def _emit_counts():
        cnt_out_ref[...] = cnt_s[...].reshape(1, K, 1)


def _assign(sums, counts, x, *, K, C, B, P, Pb, n_pb):
    return pl.pallas_call(
        functools.partial(_assign_kernel, K=K, C=C, Pb=Pb, n_b=B, n_pb=n_pb),
        grid=(B, n_pb),
        in_specs=[
            pl.BlockSpec((K, C), lambda b, pb: (0, 0)),
            pl.BlockSpec((K, 1), lambda b, pb: (0, 0)),
            pl.BlockSpec((1, C, Pb), lambda b, pb: (b, 0, pb)),
        ],
        out_specs=[
            pl.BlockSpec((1, 1, Pb), lambda b, pb: (b, 0, pb)),
            pl.BlockSpec((1, K, 1), lambda b, pb: (b, 0, 0)),
        ],
        out_shape=[
            jax.ShapeDtypeStruct((B, 1, P), jnp.int32),
            jax.ShapeDtypeStruct((B, K, 1), jnp.float32),
        ],
        scratch_shapes=[
            pltpu.VMEM((K, C), jnp.float32),
            pltpu.VMEM((K, 1), jnp.float32),
            pltpu.VMEM((K, 1), jnp.float32),
        ],
        compiler_params=pltpu.CompilerParams(
            dimension_semantics=("parallel", "arbitrary")),
    )(sums, counts, x)


@jax.jit
def kernel(image, cluster_centers):
    B, C, H, W = image.shape
    P = H * W
    K = cluster_centers.shape[0]
    x = image.reshape(B, C, P)
    flat_feats = jnp.transpose(x, (0, 2, 1)).reshape(-1, C)

    Pb = 6272 if P % 6272 == 0 else P
    n_pb = P // Pb

    sums = cluster_centers
    counts = jnp.ones((K, 1), jnp.float32)
    for it in range(_NUM_ITERS):
        seg, counts_b = _assign(sums, counts, x, K=K, C=C, B=B, P=P, Pb=Pb,
                                n_pb=n_pb)
        counts = jnp.sum(counts_b, axis=0)
        if it < _NUM_ITERS - 1:
            sums = jax.ops.segment_sum(flat_feats, seg.reshape(-1),
                                       num_segments=K)
    return seg.reshape(B, H, W)


# speedup vs baseline: 1.0019x; 1.0019x over previous
import functools

import jax
import jax.numpy as jnp
from jax.experimental import pallas as pl
from jax.experimental.pallas import tpu as pltpu

_NUM_ITERS = 10


def _assign_kernel(sums_ref, counts_ref, x_ref, out_ref, cnt_out_ref,
                   centers_s, c2_s, cnt_s, *, K, C, Pb, n_b, n_pb):
    pb = pl.program_id(1)

    @pl.when(pb == 0)
    def _compute_centers():
        cnt = counts_ref[...]
        sums = sums_ref[...]
        centers = jnp.where(cnt > 0.0,
                            sums / jnp.maximum(cnt, 1.0),
                            jnp.zeros_like(sums))
        centers_s[...] = centers
        c2_s[...] = jnp.sum(centers * centers, axis=1, keepdims=True)
        cnt_s[...] = jnp.zeros_like(cnt_s)

    centers = centers_s[...]
    x = x_ref[0]

    f2 = jnp.sum(x * x, axis=0, keepdims=True)
    cross = jnp.dot(centers, x, preferred_element_type=jnp.float32)
    d2 = f2 - 2.0 * cross + c2_s[...]

    minv = jnp.min(d2, axis=0, keepdims=True)
    kiota = jax.lax.broadcasted_iota(jnp.int32, (K, Pb), 0)
    idx = jnp.min(jnp.where(d2 == minv, kiota, K), axis=0, keepdims=True)
    out_ref[...] = idx.reshape(1, 1, Pb)

    onehot = (kiota == idx).astype(jnp.float32)
    cnt_s[...] += jnp.sum(onehot, axis=1, keepdims=True)

    @pl.when(pb == n_pb - 1)
    def _emit_counts():
        cnt_out_ref[...] = cnt_s[...].reshape(1, K, 1)


def _assign(sums, counts, x, *, K, C, B, P, Pb, n_pb):
    return pl.pallas_call(
        functools.partial(_assign_kernel, K=K, C=C, Pb=Pb, n_b=B, n_pb=n_pb),
        grid=(B, n_pb),
        in_specs=[
            pl.BlockSpec((K, C), lambda b, pb: (0, 0)),
            pl.BlockSpec((K, 1), lambda b, pb: (0, 0)),
            pl.BlockSpec((1, C, Pb), lambda b, pb: (b, 0, pb)),
        ],
        out_specs=[
            pl.BlockSpec((1, 1, Pb), lambda b, pb: (b, 0, pb)),
            pl.BlockSpec((1, K, 1), lambda b, pb: (b, 0, 0)),
        ],
        out_shape=[
            jax.ShapeDtypeStruct((B, 1, P), jnp.int32),
            jax.ShapeDtypeStruct((B, K, 1), jnp.float32),
        ],
        scratch_shapes=[
            pltpu.VMEM((K, C), jnp.float32),
            pltpu.VMEM((K, 1), jnp.float32),
            pltpu.VMEM((K, 1), jnp.float32),
        ],
        compiler_params=pltpu.CompilerParams(
            dimension_semantics=("parallel", "arbitrary")),
    )(sums, counts, x)


@jax.jit
def kernel(image, cluster_centers):
    B, C, H, W = image.shape
    P = H * W
    K = cluster_centers.shape[0]
    x = image.reshape(B, C, P)
    flat_feats = jnp.transpose(x, (0, 2, 1)).reshape(-1, C)

    Pb = 12544 if P % 12544 == 0 else P
    n_pb = P // Pb

    sums = cluster_centers
    counts = jnp.ones((K, 1), jnp.float32)
    for it in range(_NUM_ITERS):
        seg, counts_b = _assign(sums, counts, x, K=K, C=C, B=B, P=P, Pb=Pb,
                                n_pb=n_pb)
        counts = jnp.sum(counts_b, axis=0)
        if it < _NUM_ITERS - 1:
            sums = jax.ops.segment_sum(flat_feats, seg.reshape(-1),
                                       num_segments=K)
    return seg.reshape(B, H, W)


# speedup vs baseline: 1.0019x; 1.0000x over previous
import functools

import jax
import jax.numpy as jnp
from jax.experimental import pallas as pl
from jax.experimental.pallas import tpu as pltpu

_NUM_ITERS = 10


def _assign_kernel(sums_ref, counts_ref, x_ref, out_ref, cnt_out_ref,
                   centers_s, c2_s, cnt_s, *, K, C, Pb, n_pb):
    pb = pl.program_id(1)

    @pl.when(pb == 0)
    def _compute_centers():
        cnt = counts_ref[...]
        sums = sums_ref[...]
        centers = jnp.where(cnt > 0.0,
                            sums / jnp.maximum(cnt, 1.0),
                            jnp.zeros_like(sums))
        centers_s[...] = centers
        c2_s[...] = jnp.sum(centers * centers, axis=1, keepdims=True)
        cnt_s[...] = jnp.zeros_like(cnt_s)

    centers = centers_s[...]
    x = x_ref[0]

    f2 = jnp.sum(x * x, axis=0, keepdims=True)
    cross = jnp.dot(centers, x, preferred_element_type=jnp.float32)
    d2 = f2 - 2.0 * cross + c2_s[...]

    minv = jnp.min(d2, axis=0, keepdims=True)
    kiota = jax.lax.broadcasted_iota(jnp.int32, (K, Pb), 0)
    idx = jnp.min(jnp.where(d2 == minv, kiota, K), axis=0, keepdims=True)
    out_ref[...] = idx.reshape(1, 1, Pb)

    onehot = (kiota == idx).astype(jnp.float32)
    cnt_s[...] += jnp.sum(onehot, axis=1, keepdims=True)

    @pl.when(pb == n_pb - 1)
    def _emit_counts():
        cnt_out_ref[...] = cnt_s[...].reshape(1, K, 1)


def _assign(sums, counts, x, *, K, C, B, P, Pb, n_pb):
    return pl.pallas_call(
        functools.partial(_assign_kernel, K=K, C=C, Pb=Pb, n_pb=n_pb),
        grid=(B, n_pb),
        in_specs=[
            pl.BlockSpec((K, C), lambda b, pb: (0, 0)),
            pl.BlockSpec((K, 1), lambda b, pb: (0, 0)),
            pl.BlockSpec((1, C, Pb), lambda b, pb: (b, 0, pb)),
        ],
        out_specs=[
            pl.BlockSpec((1, 1, Pb), lambda b, pb: (b, 0, pb)),
            pl.BlockSpec((1, K, 1), lambda b, pb: (b, 0, 0)),
        ],
        out_shape=[
            jax.ShapeDtypeStruct((B, 1, P), jnp.int32),
            jax.ShapeDtypeStruct((B, K, 1), jnp.float32),
        ],
        scratch_shapes=[
            pltpu.VMEM((K, C), jnp.float32),
            pltpu.VMEM((K, 1), jnp.float32),
            pltpu.VMEM((K, 1), jnp.float32),
        ],
        compiler_params=pltpu.CompilerParams(
            dimension_semantics=("parallel", "arbitrary")),
    )(sums, counts, x)


@jax.jit
def kernel(image, cluster_centers):
    B, C, H, W = image.shape
    P = H * W
    K = cluster_centers.shape[0]
    x = image.reshape(B, C, P)
    flat_feats = jnp.transpose(x, (0, 2, 1)).reshape(-1, C)

    Pb = 12544 if P % 12544 == 0 else P
    n_pb = P // Pb

    sums = cluster_centers
    counts = jnp.ones((K, 1), jnp.float32)
    for it in range(_NUM_ITERS):
        seg, counts_b = _assign(sums, counts, x, K=K, C=C, B=B, P=P, Pb=Pb,
                                n_pb=n_pb)
        counts = jnp.sum(counts_b, axis=0)
        if it < _NUM_ITERS - 1:
            sums = jax.ops.segment_sum(flat_feats, seg.reshape(-1),
                                       num_segments=K)
    return seg.reshape(B, H, W)
